# baseline (speedup 1.0000x reference)
import functools

import jax
import jax.numpy as jnp
from jax import lax
from jax.experimental import pallas as pl
from jax.experimental.pallas import tpu as pltpu
from jax.experimental.pallas import tpu_sc as plsc

NC = 2
NS = 16
NW = NC * NS
LANES = 16
G = 128
D = 128
NACC = 8


def _make_sc_kernel(n_chunks, inv_count):
    mesh = plsc.VectorSubcoreMesh(core_axis_name="c", subcore_axis_name="s")
    scale = jnp.float32(0.1 * inv_count)

    @functools.partial(
        pl.kernel,
        out_type=jax.ShapeDtypeStruct((NW, LANES), jnp.float32),
        mesh=mesh,
        scratch_types=[
            pltpu.VMEM((n_chunks, 2, G), jnp.int32),
            pltpu.VMEM((G, D), jnp.float32),
            pltpu.VMEM((G, D), jnp.float32),
            pltpu.VMEM((G, D), jnp.float32),
            pltpu.VMEM((G, D), jnp.float32),
            pltpu.VMEM((LANES,), jnp.float32),
            pltpu.SemaphoreType.DMA,
            pltpu.SemaphoreType.DMA,
        ],
    )
    def k(emb_hbm, idx_hbm, out_hbm,
          idxv, srows0, drows0, srows1, drows1,
          accv, sem0, sem1):
        wid = lax.axis_index("s") * NC + lax.axis_index("c")
        srows = (srows0, srows1)
        drows = (drows0, drows1)
        sems = (sem0, sem1)

        pltpu.sync_copy(idx_hbm.at[wid], idxv)

        def start(chunk, b):
            pltpu.async_copy(emb_hbm.at[idxv.at[chunk, 0]], srows[b], sems[b])
            pltpu.async_copy(emb_hbm.at[idxv.at[chunk, 1]], drows[b], sems[b])

        def wait(chunk, b):
            pltpu.make_async_copy(emb_hbm.at[idxv.at[chunk, 0]], srows[b],
                                  sems[b]).wait()
            pltpu.make_async_copy(emb_hbm.at[idxv.at[chunk, 1]], drows[b],
                                  sems[b]).wait()

        def reduce_chunk(b, accs):
            sr = srows[b]
            dr = drows[b]

            @plsc.parallel_loop(0, G, carry=accs)
            def accs_out(i, a):
                new = []
                for j in range(NACC):
                    s = sr[i, pl.ds(j * LANES, LANES)]
                    t = dr[i, pl.ds(j * LANES, LANES)]
                    f = s - t
                    new.append(a[j] + f * f)
                return tuple(new)

            return accs_out

        start(0, 0)
        start(1, 1)

        n_pairs = n_chunks // 2

        def pair_body(t, accs):
            wait(2 * t, 0)

            @pl.when(t + 1 < n_pairs)
            def _():
                start(2 * t + 2, 0)

            wait(2 * t + 1, 1)

            @pl.when(t + 1 < n_pairs)
            def _():
                start(2 * t + 3, 1)

            return accs

        zeros = tuple(jnp.zeros((LANES,), jnp.float32) for _ in range(NACC))
        accs = lax.fori_loop(0, n_pairs, pair_body, zeros)
        acc = accs[0]
        for j in range(1, NACC):
            acc = acc + accs[j]
        accv[...] = acc * scale
        pltpu.sync_copy(accv, out_hbm.at[wid])

    return k


@jax.jit
def kernel(embeddings, edge_index):
    n_edges = edge_index.shape[1]
    chunk_span = NW * G * 2
    n_pad = ((n_edges + chunk_span - 1) // chunk_span) * chunk_span
    n_chunks = n_pad // (NW * G)

    ei = edge_index.astype(jnp.int32)
    pad = n_pad - n_edges
    src = jnp.pad(ei[0], (0, pad))
    dst = jnp.pad(ei[1], (0, pad))
    idx = jnp.stack([src.reshape(NW, n_chunks, G),
                     dst.reshape(NW, n_chunks, G)], axis=2)

    inv_count = 1.0 / (n_edges * embeddings.shape[1])
    k = _make_sc_kernel(n_chunks, inv_count)
    partials = k(embeddings, idx)
    return jnp.sum(partials)


# speedup vs baseline: 5.4679x; 5.4679x over previous
import functools

import jax
import jax.numpy as jnp
from jax import lax
from jax.experimental import pallas as pl
from jax.experimental.pallas import tpu as pltpu
from jax.experimental.pallas import tpu_sc as plsc

NC = 2
NS = 16
NW = NC * NS
LANES = 16
G = 64
NG = 32
D = 128
NACC = 8


def _make_sc_kernel(n_rows, n_groups, inv_count):
    mesh = plsc.VectorSubcoreMesh(core_axis_name="c", subcore_axis_name="s")
    scale = jnp.float32(0.1 * inv_count)

    @functools.partial(
        pl.kernel,
        out_type=jax.ShapeDtypeStruct((NW, LANES), jnp.float32),
        mesh=mesh,
        scratch_types=[
            pltpu.VMEM((NG, 2, G), jnp.int32),
            pltpu.VMEM((G, D), jnp.float32),
            pltpu.VMEM((G, D), jnp.float32),
            pltpu.VMEM((G, D), jnp.float32),
            pltpu.VMEM((G, D), jnp.float32),
            pltpu.VMEM((LANES,), jnp.float32),
            pltpu.VMEM_SHARED((n_rows, D), jnp.float32),
            pltpu.SemaphoreType.DMA,
            pltpu.SemaphoreType.DMA,
        ],
    )
    def k(emb_hbm, idx_hbm, out_hbm,
          idxg, srows0, drows0, srows1, drows1,
          accv, emb_sp, sem0, sem1):
        wid = lax.axis_index("s") * NC + lax.axis_index("c")
        sid = lax.axis_index("s")
        srows = (srows0, srows1)
        drows = (drows0, drows1)
        sems = (sem0, sem1)

        @pl.when(sid == 0)
        def _():
            pltpu.sync_copy(emb_hbm, emb_sp)

        plsc.subcore_barrier()

        def start(chunk, b):
            pltpu.async_copy(emb_sp.at[idxg.at[chunk, 0]], srows[b], sems[b])
            pltpu.async_copy(emb_sp.at[idxg.at[chunk, 1]], drows[b], sems[b])

        def wait(chunk, b):
            pltpu.make_async_copy(emb_sp.at[idxg.at[chunk, 0]], srows[b],
                                  sems[b]).wait()
            pltpu.make_async_copy(emb_sp.at[idxg.at[chunk, 1]], drows[b],
                                  sems[b]).wait()

        def reduce_chunk(b, accs):
            sr = srows[b]
            dr = drows[b]

            @plsc.parallel_loop(0, G, carry=accs)
            def accs_out(i, a):
                new = []
                for j in range(NACC):
                    s = sr[i, pl.ds(j * LANES, LANES)]
                    t = dr[i, pl.ds(j * LANES, LANES)]
                    f = s - t
                    new.append(a[j] + f * f)
                return tuple(new)

            return accs_out

        n_pairs = NG // 2
        accs = tuple(jnp.zeros((LANES,), jnp.float32) for _ in range(NACC))

        def pair_body(t, accs):
            wait(2 * t, 0)
            accs = reduce_chunk(0, accs)

            @pl.when(t + 1 < n_pairs)
            def _():
                start(2 * t + 2, 0)

            wait(2 * t + 1, 1)
            accs = reduce_chunk(1, accs)

            @pl.when(t + 1 < n_pairs)
            def _():
                start(2 * t + 3, 1)

            return accs

        for g in range(n_groups):
            pltpu.sync_copy(idx_hbm.at[wid, g], idxg)
            start(0, 0)
            start(1, 1)
            accs = lax.fori_loop(0, n_pairs, pair_body, accs)

        acc = accs[0]
        for j in range(1, NACC):
            acc = acc + accs[j]
        accv[...] = acc * scale
        pltpu.sync_copy(accv, out_hbm.at[wid])

    return k


@jax.jit
def kernel(embeddings, edge_index):
    n_edges = edge_index.shape[1]
    span = NW * G * NG
    n_pad = ((n_edges + span - 1) // span) * span
    n_groups = n_pad // span

    ei = edge_index.astype(jnp.int32)
    pad = n_pad - n_edges
    src = jnp.pad(ei[0], (0, pad))
    dst = jnp.pad(ei[1], (0, pad))
    idx = jnp.stack([src.reshape(NW, n_groups, NG, G),
                     dst.reshape(NW, n_groups, NG, G)], axis=3)

    inv_count = 1.0 / (n_edges * embeddings.shape[1])
    k = _make_sc_kernel(embeddings.shape[0], n_groups, inv_count)
    partials = k(embeddings, idx)
    return jnp.sum(partials)
